# baseline (speedup 1.0000x reference)
import jax
import jax.numpy as jnp
from jax import lax
from jax.experimental import pallas as pl
from jax.experimental.pallas import tpu as pltpu
from jax.experimental.pallas import tpu_sc as plsc

BATCH = 16384
D = 64
NC = 2
NS = 16
NW = NC * NS
B_PER_W = BATCH // NW
CHUNK = 128
N_CHUNKS = B_PER_W // CHUNK
LANES = 16
GROUPS = CHUNK // LANES
RB = 8000


def _re_body(in_ref, out_ref):
  x = in_ref[...].reshape(RB // 2, 2, D)
  out_ref[:, 0:D] = x[:, 0, :]
  out_ref[:, D:2 * D] = x[:, 1, :]


def _body(cat_i, col_i, fab_i, sp_i, sh_i,
          cat_t, col_t, fab_t, store_p,
          out,
          icat, icol, ifab, ispr, ishf,
          bcat, bcol, bfab, bsto,
          acc, sem):
  w = lax.axis_index("s") * NC + lax.axis_index("c")

  r0 = pl.multiple_of(w * N_CHUNKS, N_CHUNKS)
  pltpu.sync_copy(cat_i.at[pl.ds(r0, N_CHUNKS)], icat)
  pltpu.sync_copy(col_i.at[pl.ds(r0, N_CHUNKS)], icol)
  pltpu.sync_copy(fab_i.at[pl.ds(r0, N_CHUNKS)], ifab)
  pltpu.sync_copy(sp_i.at[pl.ds(r0, N_CHUNKS)], ispr)
  pltpu.sync_copy(sh_i.at[pl.ds(r0, N_CHUNKS)], ishf)

  for j in range(N_CHUNKS):
    d1 = pltpu.async_copy(cat_t.at[icat.at[j]], bcat, sem)
    d2 = pltpu.async_copy(col_t.at[icol.at[j]], bcol, sem)
    d3 = pltpu.async_copy(fab_t.at[ifab.at[j]], bfab, sem)
    d4 = pltpu.async_copy(store_p.at[ispr.at[j]], bsto, sem)
    d1.wait()
    d2.wait()
    d3.wait()
    d4.wait()

    def grp(g, _):
      hv = ishf[j, pl.ds(g * LANES, LANES)]
      for u in range(LANES):
        i = g * LANES + u
        h = hv[u]
        for c in range(D // LANES):
          s = pl.ds(c * LANES, LANES)
          acc[i // 2, pl.ds((i % 2) * D + c * LANES, LANES)] = (
              bcat[i, s] + bcol[i, s] + bfab[i, s]
              + bsto[i, pl.ds(h + c * LANES, LANES)])
      return 0

    lax.fori_loop(0, GROUPS, grp, 0)
    pbase = pl.multiple_of((w * B_PER_W + j * CHUNK) // 2, CHUNK // 2)
    pltpu.sync_copy(acc, out.at[pl.ds(pbase, CHUNK // 2)])


@jax.jit
def kernel(cat, col, fab, store, cat_table, col_table, fab_table, store_table):
  kr = pl.pallas_call(
      _re_body,
      grid=(1000000 // RB,),
      in_specs=[pl.BlockSpec((RB, D), lambda i: (i, 0))],
      out_specs=pl.BlockSpec((RB // 2, 2 * D), lambda i: (i, 0)),
      out_shape=jax.ShapeDtypeStruct((500000, 2 * D), jnp.float32),
  )
  store_pairs = kr(store_table)

  mesh = plsc.VectorSubcoreMesh(core_axis_name="c", subcore_axis_name="s")
  k = pl.kernel(
      _body,
      out_type=jax.ShapeDtypeStruct((BATCH // 2, 2 * D), jnp.float32),
      mesh=mesh,
      scratch_types=[
          pltpu.VMEM((N_CHUNKS, CHUNK), jnp.int32),
          pltpu.VMEM((N_CHUNKS, CHUNK), jnp.int32),
          pltpu.VMEM((N_CHUNKS, CHUNK), jnp.int32),
          pltpu.VMEM((N_CHUNKS, CHUNK), jnp.int32),
          pltpu.VMEM((N_CHUNKS, CHUNK), jnp.int32),
          pltpu.VMEM((CHUNK, 2 * D), jnp.float32),
          pltpu.VMEM((CHUNK, 2 * D), jnp.float32),
          pltpu.VMEM((CHUNK, 2 * D), jnp.float32),
          pltpu.VMEM((CHUNK, 2 * D), jnp.float32),
          pltpu.VMEM((CHUNK // 2, 2 * D), jnp.float32),
          pltpu.SemaphoreType.DMA,
      ],
  )
  shape2 = (NW * N_CHUNKS, CHUNK)
  pad = ((0, 0), (0, D))
  catp = jnp.pad(cat_table, pad)
  colp = jnp.pad(col_table, pad)
  fabp = jnp.pad(fab_table, pad)
  spair = (store >> 1).reshape(shape2)
  shalf = ((store & 1) * D).reshape(shape2)
  p = k(cat.reshape(shape2), col.reshape(shape2), fab.reshape(shape2),
        spair, shalf, catp, colp, fabp, store_pairs)
  return p.reshape(BATCH, D)


# speedup vs baseline: 1.9800x; 1.9800x over previous
import jax
import jax.numpy as jnp
from jax import lax
from jax.experimental import pallas as pl
from jax.experimental.pallas import tpu as pltpu
from jax.experimental.pallas import tpu_sc as plsc

BATCH = 16384
D = 64
NC = 2
NS = 16
NW = NC * NS
B_PER_W = BATCH // NW
CHUNK = 128
N_CHUNKS = B_PER_W // CHUNK
LANES = 16
GROUPS = CHUNK // LANES
PAIR_ROWS = CHUNK // 2


def _body_small(cat_i, col_i, fab_i,
                cat_t, col_t, fab_t,
                p_out,
                icat, icol, ifab,
                bcat, bcol, bfab,
                pacc, sem):
  w = lax.axis_index("s") * NC + lax.axis_index("c")

  pltpu.sync_copy(cat_i.at[w], icat)
  pltpu.sync_copy(col_i.at[w], icol)
  pltpu.sync_copy(fab_i.at[w], ifab)

  for j in range(N_CHUNKS):
    d1 = pltpu.async_copy(cat_t.at[icat.at[j]], bcat, sem)
    d2 = pltpu.async_copy(col_t.at[icol.at[j]], bcol, sem)
    d3 = pltpu.async_copy(fab_t.at[ifab.at[j]], bfab, sem)
    d1.wait()
    d2.wait()
    d3.wait()

    def pair(p, _):
      for half in range(2):
        i = 2 * p + half
        for c in range(D // LANES):
          s = pl.ds(c * LANES, LANES)
          pacc[p, pl.ds(half * D + c * LANES, LANES)] = (
              bcat[i, s] + bcol[i, s] + bfab[i, s])
      return 0

    lax.fori_loop(0, PAIR_ROWS, pair, 0)
    pbase = pl.multiple_of((w * B_PER_W + j * CHUNK) // 2, PAIR_ROWS)
    pltpu.sync_copy(pacc, p_out.at[pl.ds(pbase, PAIR_ROWS)])


def _body_store(store_i, store_t, p_in, out,
                isto, bsto, pacc, acc, sem):
  w = lax.axis_index("s") * NC + lax.axis_index("c")
  base = pl.multiple_of(w * B_PER_W, B_PER_W)

  pltpu.sync_copy(store_i.at[pl.ds(base, B_PER_W)], isto)

  for j in range(N_CHUNKS):
    def grp(g, _):
      vec = isto[pl.ds(j * CHUNK + g * LANES, LANES)]
      for u in range(LANES):
        pltpu.async_copy(store_t.at[vec[u]], bsto.at[g * LANES + u], sem)
      return 0

    lax.fori_loop(0, GROUPS, grp, 0)
    pbase = pl.multiple_of((base + j * CHUNK) // 2, PAIR_ROWS)
    dp = pltpu.async_copy(p_in.at[pl.ds(pbase, PAIR_ROWS)], pacc, sem)
    dp.wait()
    pltpu.make_async_copy(store_t.at[pl.ds(0, CHUNK)], bsto, sem).wait()

    def pair(p, _):
      for half in range(2):
        i = 2 * p + half
        for c in range(D // LANES):
          s = pl.ds(c * LANES, LANES)
          acc[i, s] = bsto[i, s] + pacc[p, pl.ds(half * D + c * LANES, LANES)]
      return 0

    lax.fori_loop(0, PAIR_ROWS, pair, 0)
    pltpu.sync_copy(acc, out.at[pl.ds(base + j * CHUNK, CHUNK)])


@jax.jit
def kernel(cat, col, fab, store, cat_table, col_table, fab_table, store_table):
  mesh = plsc.VectorSubcoreMesh(core_axis_name="c", subcore_axis_name="s")

  kb = pl.kernel(
      _body_small,
      out_type=jax.ShapeDtypeStruct((BATCH // 2, 2 * D), jnp.float32),
      mesh=mesh,
      compiler_params=pltpu.CompilerParams(use_tc_tiling_on_sc=False),
      scratch_types=[
          pltpu.VMEM((N_CHUNKS, CHUNK), jnp.int32),
          pltpu.VMEM((N_CHUNKS, CHUNK), jnp.int32),
          pltpu.VMEM((N_CHUNKS, CHUNK), jnp.int32),
          pltpu.VMEM((CHUNK, D), jnp.float32),
          pltpu.VMEM((CHUNK, D), jnp.float32),
          pltpu.VMEM((CHUNK, D), jnp.float32),
          pltpu.VMEM((PAIR_ROWS, 2 * D), jnp.float32),
          pltpu.SemaphoreType.DMA,
      ],
  )
  shape3 = (NW, N_CHUNKS, CHUNK)
  p = kb(cat.reshape(shape3), col.reshape(shape3), fab.reshape(shape3),
         cat_table, col_table, fab_table)

  ka = pl.kernel(
      _body_store,
      out_type=jax.ShapeDtypeStruct((BATCH, D), jnp.float32),
      mesh=mesh,
      scratch_types=[
          pltpu.VMEM((B_PER_W,), jnp.int32),
          pltpu.VMEM((CHUNK, D), jnp.float32),
          pltpu.VMEM((PAIR_ROWS, 2 * D), jnp.float32),
          pltpu.VMEM((CHUNK, D), jnp.float32),
          pltpu.SemaphoreType.DMA,
      ],
  )
  return ka(store, store_table, p)
